# baseline (speedup 1.0000x reference)
import functools

import jax
import jax.numpy as jnp
from jax.experimental import pallas as pl
from jax.experimental.pallas import tpu as pltpu

RESO = 128
PADDING = 0.1
NSEG = RESO * RESO


def _head_body(p_ref, inv_ref, wp_ref, bp_ref, w0_ref, b0_ref, w1_ref, b1_ref,
               ws_ref, net_ref, idx_ref):
    p = p_ref[...]
    inv = inv_ref[0]
    x0 = jnp.clip(p[:, 0] * inv + 0.5, 0.0, 1.0 - 1e-05)
    x2 = jnp.clip(p[:, 2] * inv + 0.5, 0.0, 1.0 - 1e-05)
    ix = jnp.floor(x0 * RESO).astype(jnp.int32)
    iy = jnp.floor(x2 * RESO).astype(jnp.int32)
    idx_ref[...] = (ix + RESO * iy).reshape(idx_ref.shape)
    net = jnp.dot(p, wp_ref[...].T, preferred_element_type=jnp.float32) + bp_ref[...]
    h = jnp.dot(jax.nn.relu(net), w0_ref[...].T,
                preferred_element_type=jnp.float32) + b0_ref[...]
    dx = jnp.dot(jax.nn.relu(h), w1_ref[...].T,
                 preferred_element_type=jnp.float32) + b1_ref[...]
    net_ref[...] = jnp.dot(net, ws_ref[...].T,
                           preferred_element_type=jnp.float32) + dx


def _resblock_body(net_ref, pooled_ref, w0_ref, b0_ref, w1_ref, b1_ref,
                   ws_ref, out_ref):
    x = jnp.concatenate([net_ref[...], pooled_ref[...]], axis=1)
    h = jnp.dot(jax.nn.relu(x), w0_ref[...].T,
                preferred_element_type=jnp.float32) + b0_ref[...]
    dx = jnp.dot(jax.nn.relu(h), w1_ref[...].T,
                 preferred_element_type=jnp.float32) + b1_ref[...]
    out_ref[...] = jnp.dot(x, ws_ref[...].T,
                           preferred_element_type=jnp.float32) + dx


def _proj_body(net_ref, wc_ref, bc_ref, out_ref):
    out_ref[...] = jnp.dot(net_ref[...], wc_ref[...].T,
                           preferred_element_type=jnp.float32) + bc_ref[...]


def _full(shape):
    nd = len(shape)
    return pl.BlockSpec(shape, lambda i: (0,) * nd)


def kernel(p, scale, fc_pos_w, fc_pos_b, b_fc0_w, b_fc0_b, b_fc1_w, b_fc1_b,
           b_sc_w, fc_c_w, fc_c_b):
    B, T, D = p.shape
    N = B * T
    P = 8000
    grid = (N // P,)
    c_dim = fc_c_w.shape[0]

    pf = p.reshape(N, D)
    inv = jnp.full((1,), 1.0 / (scale * (1.0 + PADDING + 1e-05)), jnp.float32)

    h = b_fc0_w.shape[1]
    net, idx = pl.pallas_call(
        _head_body,
        grid=grid,
        in_specs=[
            pl.BlockSpec((P, D), lambda i: (i, 0)),
            _full((1,)),
            _full(fc_pos_w.shape), _full(fc_pos_b.shape),
            _full(b_fc0_w.shape[1:]), _full(b_fc0_b.shape[1:]),
            _full(b_fc1_w.shape[1:]), _full(b_fc1_b.shape[1:]),
            _full(b_sc_w.shape[1:]),
        ],
        out_specs=[
            pl.BlockSpec((P, h), lambda i: (i, 0)),
            pl.BlockSpec((1, 1, P), lambda i: (i, 0, 0)),
        ],
        out_shape=[
            jax.ShapeDtypeStruct((N, h), jnp.float32),
            jax.ShapeDtypeStruct((N // P, 1, P), jnp.int32),
        ],
    )(pf, inv, fc_pos_w, fc_pos_b, b_fc0_w[0], b_fc0_b[0], b_fc1_w[0],
      b_fc1_b[0], b_sc_w[0])

    idx = idx.reshape(N)
    flat_idx = idx + (jnp.arange(N, dtype=jnp.int32) // T) * NSEG

    resblock = pl.pallas_call(
        _resblock_body,
        grid=grid,
        in_specs=[
            pl.BlockSpec((P, h), lambda i: (i, 0)),
            pl.BlockSpec((P, h), lambda i: (i, 0)),
            _full(b_fc0_w.shape[1:]), _full(b_fc0_b.shape[1:]),
            _full(b_fc1_w.shape[1:]), _full(b_fc1_b.shape[1:]),
            _full(b_sc_w.shape[1:]),
        ],
        out_specs=pl.BlockSpec((P, h), lambda i: (i, 0)),
        out_shape=jax.ShapeDtypeStruct((N, h), jnp.float32),
    )

    for i in range(1, b_fc0_w.shape[0]):
        seg = jax.ops.segment_max(net, flat_idx, num_segments=B * NSEG)
        seg = jnp.where(jnp.isfinite(seg), seg, 0.0)
        pooled = seg[flat_idx]
        net = resblock(net, pooled, b_fc0_w[i], b_fc0_b[i], b_fc1_w[i],
                       b_fc1_b[i], b_sc_w[i])

    c = pl.pallas_call(
        _proj_body,
        grid=grid,
        in_specs=[
            pl.BlockSpec((P, h), lambda i: (i, 0)),
            _full(fc_c_w.shape), _full(fc_c_b.shape),
        ],
        out_specs=pl.BlockSpec((P, c_dim), lambda i: (i, 0)),
        out_shape=jax.ShapeDtypeStruct((N, c_dim), jnp.float32),
    )(net, fc_c_w, fc_c_b)

    sums = jax.ops.segment_sum(c, flat_idx, num_segments=B * NSEG)
    cnt = jax.ops.segment_sum(jnp.ones((N, 1), jnp.float32), flat_idx,
                              num_segments=B * NSEG)
    fea = sums / jnp.maximum(cnt, 1.0)
    fea = fea.reshape(B, NSEG, c_dim).transpose(0, 2, 1)
    return fea.reshape(B, c_dim, RESO, RESO)


# speedup vs baseline: 1.8550x; 1.8550x over previous
import functools

import jax
import jax.numpy as jnp
from jax import lax
from jax.experimental import pallas as pl
from jax.experimental.pallas import tpu as pltpu
from jax.experimental.pallas import tpu_sc as plsc

RESO = 128
PADDING = 0.1
NSEG = RESO * RESO

NCORE = 2
NSUB = 16

FQ = 4
CHUNK = 4096
NSEG2 = 16416
T_PAD = 102400


def _pool_body(net_hbm, idx_hbm, pooled_hbm, tbl, idxbuf, valbuf, outbuf):
    c = lax.axis_index("c")
    s = lax.axis_index("s")
    w = s * NCORE + c
    b = w // 8
    q = w % 8
    nch = T_PAD // CHUNK
    nvec = CHUNK // 16

    neg = jnp.full((16,), -jnp.inf, jnp.float32)

    def initb(i, _):
        for u in range(8):
            tbl[pl.ds((i * 8 + u) * 16, 16)] = neg
        return 0

    lax.fori_loop(0, (FQ * NSEG2) // (16 * 8), initb, 0, unroll=False)

    for j in range(nch):
        base = b * T_PAD + j * CHUNK
        pltpu.sync_copy(idx_hbm.at[pl.ds(base, CHUNK)], idxbuf)
        pltpu.sync_copy(net_hbm.at[q, :, pl.ds(base, CHUNK)], valbuf)

        def upd(v, _):
            o = v * 16
            iv = idxbuf[pl.ds(o, 16)]
            bad = jnp.zeros((16,), jnp.bool_)
            for f in range(FQ):
                off = iv + f * NSEG2
                val = valbuf[f, pl.ds(o, 16)]
                cur = plsc.load_gather(tbl, [off])
                plsc.store_scatter(tbl, [off], jnp.maximum(cur, val))
                chk = plsc.load_gather(tbl, [off])
                bad = jnp.logical_or(bad, val > chk)
            nb = jnp.sum(bad.astype(jnp.int32))

            def slow():
                for f in range(FQ):
                    off = iv + f * NSEG2
                    val = valbuf[f, pl.ds(o, 16)]

                    def cond_fn(k):
                        return k > 0

                    def body_fn(k):
                        cur2 = plsc.load_gather(tbl, [off])
                        need = val > cur2
                        plsc.store_scatter(tbl, [off],
                                           jnp.maximum(cur2, val), mask=need)
                        chk2 = plsc.load_gather(tbl, [off])
                        return jnp.sum((val > chk2).astype(jnp.int32))

                    lax.while_loop(cond_fn, body_fn, jnp.int32(1))

            pl.when(nb > 0)(slow)
            return 0

        lax.fori_loop(0, nvec, upd, 0, unroll=False)

    for j in range(nch):
        base = b * T_PAD + j * CHUNK
        pltpu.sync_copy(idx_hbm.at[pl.ds(base, CHUNK)], idxbuf)

        def gat(v, _):
            o = v * 16
            iv = idxbuf[pl.ds(o, 16)]
            for f in range(FQ):
                off = iv + f * NSEG2
                outbuf[f, pl.ds(o, 16)] = plsc.load_gather(tbl, [off])
            return 0

        lax.fori_loop(0, nvec, gat, 0, unroll=False)
        pltpu.sync_copy(outbuf, pooled_hbm.at[q, :, pl.ds(base, CHUNK)])


def _make_pool(n_pad):
    mesh = plsc.VectorSubcoreMesh(core_axis_name="c", subcore_axis_name="s",
                                  num_cores=NCORE, num_subcores=NSUB)
    return pl.kernel(
        _pool_body,
        out_type=jax.ShapeDtypeStruct((8, FQ, n_pad), jnp.float32),
        mesh=mesh,
        compiler_params=pltpu.CompilerParams(needs_layout_passes=False),
        scratch_types=[
            pltpu.VMEM((FQ * NSEG2,), jnp.float32),
            pltpu.VMEM((CHUNK,), jnp.int32),
            pltpu.VMEM((FQ, CHUNK), jnp.float32),
            pltpu.VMEM((FQ, CHUNK), jnp.float32),
        ],
    )


def _head_body(T, P, p_ref, inv_ref, wp_ref, bp_ref, w0_ref, b0_ref, w1_ref,
               b1_ref, ws_ref, net_ref, idx_ref):
    p = p_ref[...]
    inv = inv_ref[0]
    col = pl.program_id(0) * P + lax.broadcasted_iota(jnp.int32, (P,), 0)
    padm = (col % T_PAD) >= T
    x0 = jnp.clip(p[:, 0] * inv + 0.5, 0.0, 1.0 - 1e-05)
    x2 = jnp.clip(p[:, 2] * inv + 0.5, 0.0, 1.0 - 1e-05)
    ix = jnp.floor(x0 * RESO).astype(jnp.int32)
    iy = jnp.floor(x2 * RESO).astype(jnp.int32)
    idx = jnp.where(padm, NSEG, ix + RESO * iy)
    idx_ref[...] = idx.reshape(idx_ref.shape)
    wp = wp_ref[...]
    net = (wp[:, 0:1] * p[:, 0].reshape(1, P)
           + wp[:, 1:2] * p[:, 1].reshape(1, P)
           + wp[:, 2:3] * p[:, 2].reshape(1, P))
    net = net + bp_ref[...][:, None]
    h1 = jnp.dot(w0_ref[...], jax.nn.relu(net),
                 preferred_element_type=jnp.float32) + b0_ref[...][:, None]
    dx = jnp.dot(w1_ref[...], jax.nn.relu(h1),
                 preferred_element_type=jnp.float32) + b1_ref[...][:, None]
    out = jnp.dot(ws_ref[...], net,
                  preferred_element_type=jnp.float32) + dx
    out = jnp.where(padm[None, :], 0.0, out)
    net_ref[...] = out.reshape(net_ref.shape)


def _resblock_body(net_ref, pooled_ref, w0_ref, b0_ref, w1_ref, b1_ref,
                   ws_ref, out_ref):
    P = net_ref.shape[-1]
    net = net_ref[...].reshape(32, P)
    pooled = pooled_ref[...].reshape(32, P)
    x = jnp.concatenate([net, pooled], axis=0)
    h1 = jnp.dot(w0_ref[...], jax.nn.relu(x),
                 preferred_element_type=jnp.float32) + b0_ref[...][:, None]
    dx = jnp.dot(w1_ref[...], jax.nn.relu(h1),
                 preferred_element_type=jnp.float32) + b1_ref[...][:, None]
    out = jnp.dot(ws_ref[...], x,
                  preferred_element_type=jnp.float32) + dx
    out_ref[...] = out.reshape(out_ref.shape)


def _resblock_proj_body(net_ref, pooled_ref, w0_ref, b0_ref, w1_ref, b1_ref,
                        ws_ref, wc_ref, bc_ref, c_ref):
    P = net_ref.shape[-1]
    net = net_ref[...].reshape(32, P)
    pooled = pooled_ref[...].reshape(32, P)
    x = jnp.concatenate([net, pooled], axis=0)
    h1 = jnp.dot(w0_ref[...], jax.nn.relu(x),
                 preferred_element_type=jnp.float32) + b0_ref[...][:, None]
    dx = jnp.dot(w1_ref[...], jax.nn.relu(h1),
                 preferred_element_type=jnp.float32) + b1_ref[...][:, None]
    net2 = jnp.dot(ws_ref[...], x,
                   preferred_element_type=jnp.float32) + dx
    cc = jnp.dot(wc_ref[...], net2,
                 preferred_element_type=jnp.float32) + bc_ref[...][:, None]
    c_ref[...] = cc.T


def _full(shape):
    nd = len(shape)
    return pl.BlockSpec(shape, lambda i: (0,) * nd)


def kernel(p, scale, fc_pos_w, fc_pos_b, b_fc0_w, b_fc0_b, b_fc1_w, b_fc1_b,
           b_sc_w, fc_c_w, fc_c_b):
    B, T, D = p.shape
    n_pad = B * T_PAD
    P = 16384
    grid = (n_pad // P,)
    c_dim = fc_c_w.shape[0]
    h = b_fc0_w.shape[1]

    pf = jnp.pad(p, ((0, 0), (0, T_PAD - T), (0, 0))).reshape(n_pad, D)
    inv = jnp.full((1,), 1.0 / (scale * (1.0 + PADDING + 1e-05)), jnp.float32)

    fm_spec = pl.BlockSpec((8, FQ, P), lambda i: (0, 0, i))

    net3, idx3 = pl.pallas_call(
        functools.partial(_head_body, T, P),
        grid=grid,
        in_specs=[
            pl.BlockSpec((P, D), lambda i: (i, 0)),
            _full((1,)),
            _full(fc_pos_w.shape), _full(fc_pos_b.shape),
            _full(b_fc0_w.shape[1:]), _full(b_fc0_b.shape[1:]),
            _full(b_fc1_w.shape[1:]), _full(b_fc1_b.shape[1:]),
            _full(b_sc_w.shape[1:]),
        ],
        out_specs=[
            fm_spec,
            pl.BlockSpec((1, 1, P), lambda i: (i, 0, 0)),
        ],
        out_shape=[
            jax.ShapeDtypeStruct((8, FQ, n_pad), jnp.float32),
            jax.ShapeDtypeStruct((n_pad // P, 1, P), jnp.int32),
        ],
    )(pf, inv, fc_pos_w, fc_pos_b, b_fc0_w[0], b_fc0_b[0], b_fc1_w[0],
      b_fc1_b[0], b_sc_w[0])

    idx = idx3.reshape(n_pad)
    pool = _make_pool(n_pad)

    resblock = pl.pallas_call(
        _resblock_body,
        grid=grid,
        in_specs=[
            fm_spec,
            fm_spec,
            _full(b_fc0_w.shape[1:]), _full(b_fc0_b.shape[1:]),
            _full(b_fc1_w.shape[1:]), _full(b_fc1_b.shape[1:]),
            _full(b_sc_w.shape[1:]),
        ],
        out_specs=fm_spec,
        out_shape=jax.ShapeDtypeStruct((8, FQ, n_pad), jnp.float32),
    )

    nb = b_fc0_w.shape[0]
    for i in range(1, nb - 1):
        pooled3 = pool(net3, idx)
        net3 = resblock(net3, pooled3, b_fc0_w[i], b_fc0_b[i], b_fc1_w[i],
                        b_fc1_b[i], b_sc_w[i])

    pooled3 = pool(net3, idx)
    c = pl.pallas_call(
        _resblock_proj_body,
        grid=grid,
        in_specs=[
            fm_spec,
            fm_spec,
            _full(b_fc0_w.shape[1:]), _full(b_fc0_b.shape[1:]),
            _full(b_fc1_w.shape[1:]), _full(b_fc1_b.shape[1:]),
            _full(b_sc_w.shape[1:]),
            _full(fc_c_w.shape), _full(fc_c_b.shape),
        ],
        out_specs=pl.BlockSpec((P, c_dim), lambda i: (i, 0)),
        out_shape=jax.ShapeDtypeStruct((n_pad, c_dim), jnp.float32),
    )(net3, pooled3, b_fc0_w[nb - 1], b_fc0_b[nb - 1], b_fc1_w[nb - 1],
      b_fc1_b[nb - 1], b_sc_w[nb - 1], fc_c_w, fc_c_b)

    flat_idx = idx + (jnp.arange(n_pad, dtype=jnp.int32) // T_PAD) * (NSEG + 1)
    sums = jax.ops.segment_sum(c, flat_idx, num_segments=B * (NSEG + 1))
    cnt = jax.ops.segment_sum(jnp.ones((n_pad, 1), jnp.float32), flat_idx,
                              num_segments=B * (NSEG + 1))
    fea = sums / jnp.maximum(cnt, 1.0)
    fea = fea.reshape(B, NSEG + 1, c_dim)[:, :NSEG].transpose(0, 2, 1)
    return fea.reshape(B, c_dim, RESO, RESO)
